# baseline (speedup 1.0000x reference)
import jax
import jax.numpy as jnp
from jax import lax
from jax.experimental import pallas as pl
from jax.experimental.pallas import tpu as pltpu
from jax.experimental.pallas import tpu_sc as plsc

B, S, D, H, E, K, L, V, F = 1, 2048, 768, 12, 8, 2, 2, 32000, 1536
DH = D // H
BT = 512
NT = S // BT
BV = 1280
NV = V // BV
NSC = 32
TPW = S // NSC
BTM = 256
NB = 24
NBP = NB * BTM
JW = (K * S) // NSC

_BF = jnp.bfloat16
_F32 = jnp.float32


def _emb_gather(ids, table):
    mesh = plsc.VectorSubcoreMesh(core_axis_name="c", subcore_axis_name="s")

    def body(ids_hbm, table_hbm, out_hbm, idx_v, rows_v, sem):
        wid = lax.axis_index("s") * 2 + lax.axis_index("c")
        base = wid * TPW
        pltpu.sync_copy(ids_hbm.at[pl.ds(base, TPW)], idx_v)
        pltpu.async_copy(table_hbm.at[idx_v], rows_v, sem).wait()
        pltpu.sync_copy(rows_v, out_hbm.at[pl.ds(base, TPW)])

    call = pl.kernel(
        body,
        mesh=mesh,
        out_type=jax.ShapeDtypeStruct((S, D), _F32),
        scratch_types=[
            pltpu.VMEM((TPW,), jnp.int32),
            pltpu.VMEM((TPW, D), _F32),
            pltpu.SemaphoreType.DMA,
        ],
    )
    return call(ids, table)


def _ln(x, g, b):
    mu = jnp.mean(x, axis=-1, keepdims=True)
    var = jnp.mean((x - mu) ** 2, axis=-1, keepdims=True)
    return (x - mu) * lax.rsqrt(var + 1e-5) * g + b


def _qkv(x, wq, bq, wk, bk, wv, bv, pre=None):
    mode = pre[0] if pre is not None else None

    def body(*refs):
        if mode == 'add':
            (x_ref, p_ref, wq_ref, bq_ref, wk_ref, bk_ref, wv_ref, bv_ref,
             q_ref, k_ref, v_ref, xs_ref) = refs
            xs = x_ref[...] + p_ref[...]
        elif mode == 'combine':
            (x_ref, y0_ref, y1_ref, tg_ref, wq_ref, bq_ref, wk_ref,
             bk_ref, wv_ref, bv_ref, q_ref, k_ref, v_ref, xs_ref) = refs
            tg = tg_ref[...]
            xs = (x_ref[...] + tg[:, 0:1] * y0_ref[...]
                  + tg[:, 1:2] * y1_ref[...])
        else:
            (x_ref, wq_ref, bq_ref, wk_ref, bk_ref, wv_ref, bv_ref,
             q_ref, k_ref, v_ref, xs_ref) = refs
            xs = x_ref[...]
        xs_ref[...] = xs
        xb = xs.astype(_BF)
        for w_ref, b_ref, o_ref, scale in ((wq_ref, bq_ref, q_ref, 0.125),
                                           (wk_ref, bk_ref, k_ref, 1.0),
                                           (wv_ref, bv_ref, v_ref, 1.0)):
            y = jnp.dot(xb, w_ref[...].astype(_BF),
                        preferred_element_type=_F32) + b_ref[...]
            o_ref[...] = (y * scale if scale != 1.0 else y).astype(_BF)

    xspec = pl.BlockSpec((BT, D), lambda t: (t, 0))
    wspec = pl.BlockSpec((D, D), lambda t: (0, 0))
    bspec = pl.BlockSpec((1, D), lambda t: (0, 0))
    in_specs = [xspec]
    args = [x]
    if mode == 'add':
        in_specs += [xspec]
        args += [pre[1]]
    elif mode == 'combine':
        in_specs += [xspec, xspec, pl.BlockSpec((BT, K), lambda t: (t, 0))]
        args += list(pre[1:])
    in_specs += [wspec, bspec, wspec, bspec, wspec, bspec]
    args += [wq, bq, wk, bk, wv, bv]
    return pl.pallas_call(
        body, grid=(NT,),
        in_specs=in_specs,
        out_specs=[xspec, xspec, xspec, xspec],
        out_shape=[jax.ShapeDtypeStruct((S, D), _BF)] * 3
        + [jax.ShapeDtypeStruct((S, D), _F32)],
    )(*args)


def _attention(q, k, v):
    NH = 4

    def body(q_ref, k_ref, v_ref, o_ref):
        t = pl.program_id(1)
        ri = lax.broadcasted_iota(jnp.int32, (BT, BT), 0)
        ci = lax.broadcasted_iota(jnp.int32, (BT, BT), 1)
        diag_ok = ci <= ri

        m0 = jnp.full((BT, 1), -1e30, _F32)
        l0 = jnp.zeros((BT, 1), _F32)
        a0 = jnp.zeros((BT, DH), _F32)

        def step(c, carry):
            on_diag = c == t
            out = []
            for i in range(NH):
                m, l, acc = carry[i]
                kc = k_ref[pl.ds(c * BT, BT), i * DH:(i + 1) * DH]
                vc = v_ref[pl.ds(c * BT, BT), i * DH:(i + 1) * DH]
                s = lax.dot_general(q_ref[:, i * DH:(i + 1) * DH], kc,
                                    (((1,), (1,)), ((), ())),
                                    preferred_element_type=_F32)
                s = lax.cond(on_diag,
                             lambda s=s: jnp.where(diag_ok, s, -1e9),
                             lambda s=s: s)
                mn = jnp.maximum(m, jnp.max(s, axis=-1, keepdims=True))
                p = jnp.exp(s - mn)
                corr = jnp.exp(m - mn)
                ln = l * corr + jnp.sum(p, axis=-1, keepdims=True)
                accn = acc * corr + jnp.dot(p.astype(_BF), vc,
                                            preferred_element_type=_F32)
                out.append((mn, ln, accn))
            return tuple(out)

        carry = lax.fori_loop(0, t + 1, step, tuple((m0, l0, a0)
                                                    for _ in range(NH)))
        o_ref[...] = jnp.concatenate([acc / l for (m, l, acc) in carry],
                                     axis=1).astype(_BF)

    return pl.pallas_call(
        body, grid=(H // NH, NT),
        in_specs=[pl.BlockSpec((BT, NH * DH), lambda h, t: (t, h)),
                  pl.BlockSpec((S, NH * DH), lambda h, t: (0, h)),
                  pl.BlockSpec((S, NH * DH), lambda h, t: (0, h))],
        out_specs=pl.BlockSpec((BT, NH * DH), lambda h, t: (t, h)),
        out_shape=jax.ShapeDtypeStruct((S, D), _BF))(q, k, v)


def _oproj_lns(attn, wo, bo, x, g1, b1, g2, b2, wr, br):
    def body(a_ref, wo_ref, bo_ref, x_ref, g1_ref, b1_ref, g2_ref, b2_ref,
             wr_ref, br_ref, x1_ref, h_ref, ti_ref, tg_ref):
        a = jnp.dot(a_ref[...].astype(_BF), wo_ref[...].astype(_BF),
                    preferred_element_type=_F32)
        a = a + bo_ref[...] + x_ref[...]
        x1 = _ln(a, g1_ref[...], b1_ref[...])
        x1_ref[...] = x1
        hh = _ln(x1, g2_ref[...], b2_ref[...])
        h_ref[...] = hh
        rl = jnp.dot(hh, wr_ref[...],
                     preferred_element_type=_F32) + br_ref[...]
        iota = lax.broadcasted_iota(jnp.int32, (BT, E), 1)
        v1 = jnp.max(rl, axis=-1, keepdims=True)
        i1 = jnp.min(jnp.where(rl == v1, iota, E), axis=-1, keepdims=True)
        m1 = iota == i1
        rl2 = jnp.where(m1, -jnp.inf, rl)
        v2 = jnp.max(rl2, axis=-1, keepdims=True)
        i2 = jnp.min(jnp.where(rl2 == v2, iota, E), axis=-1, keepdims=True)
        m2 = iota == i2
        z = jnp.exp(v2 - v1)
        gate1 = 1.0 / (1.0 + z)
        gate2 = 1.0 - gate1
        ti_ref[...] = jnp.concatenate([i1, i2], axis=1)
        tg_ref[...] = jnp.concatenate([gate1, gate2], axis=1)

    xspec = pl.BlockSpec((BT, D), lambda t: (t, 0))
    wspec = pl.BlockSpec((D, D), lambda t: (0, 0))
    vspec = pl.BlockSpec((1, D), lambda t: (0, 0))
    tspec = pl.BlockSpec((BT, K), lambda t: (t, 0))
    return pl.pallas_call(
        body, grid=(NT,),
        in_specs=[xspec, wspec, vspec, xspec, vspec, vspec, vspec, vspec,
                  pl.BlockSpec((D, E), lambda t: (0, 0)),
                  pl.BlockSpec((1, E), lambda t: (0, 0))],
        out_specs=[xspec, xspec, tspec, tspec],
        out_shape=[jax.ShapeDtypeStruct((S, D), _F32),
                   jax.ShapeDtypeStruct((S, D), _F32),
                   jax.ShapeDtypeStruct((S, K), jnp.int32),
                   jax.ShapeDtypeStruct((S, K), _F32)],
    )(attn, wo, bo, x, g1, b1, g2, b2, wr, br)


def _route_meta(topi):
    CH = 512

    def body(ti_ref, pos_ref, be_ref, nb_ref):
        ti = ti_ref[...]
        lane_e = lax.broadcasted_iota(jnp.int32, (S, E), 1)
        oh = [jnp.where(lane_e == ti[:, kk:kk + 1], 1.0, 0.0)
              for kk in range(K)]
        lt = (lax.broadcasted_iota(jnp.int32, (CH, CH), 1) <
              lax.broadcasted_iota(jnp.int32, (CH, CH), 0)).astype(_F32)
        carry = jnp.zeros((1, E), _F32)
        ranks = []
        for kk in range(K):
            rk = []
            for ch in range(S // CH):
                blk = oh[kk][ch * CH:(ch + 1) * CH]
                rk.append(jnp.dot(lt, blk, preferred_element_type=_F32)
                          + carry)
                carry = carry + jnp.sum(blk, axis=0, keepdims=True)
            ranks.append(jnp.concatenate(rk, axis=0))
        counts = carry
        nblk = jnp.floor((counts + (BTM - 1)) / BTM)
        ut8 = (lax.broadcasted_iota(jnp.int32, (E, E), 0) <=
               lax.broadcasted_iota(jnp.int32, (E, E), 1)).astype(_F32)
        cumblk = jnp.dot(nblk, ut8, preferred_element_type=_F32)
        pad_off = (cumblk - nblk) * BTM
        poss = [jnp.sum(oh[kk] * (pad_off + ranks[kk]), axis=1,
                        keepdims=True) for kk in range(K)]
        pos_ref[...] = jnp.concatenate(poss, axis=1).astype(jnp.int32)
        cumblk_i = cumblk.astype(jnp.int32)
        rows_b = lax.broadcasted_iota(jnp.int32, (NB, E), 0)
        be = jnp.sum(jnp.where(rows_b >= cumblk_i, 1, 0), axis=1,
                     keepdims=True)
        be_ref[...] = jnp.minimum(be, E - 1)
        nb_ref[...] = jnp.sum(
            jnp.where(lax.broadcasted_iota(jnp.int32, (1, E), 1) == E - 1,
                      cumblk_i, 0), axis=1, keepdims=True)

    return pl.pallas_call(
        body, grid=(1,),
        in_specs=[pl.BlockSpec((S, K), lambda i: (0, 0))],
        out_specs=[pl.BlockSpec((S, K), lambda i: (0, 0)),
                   pl.BlockSpec((NB, 1), lambda i: (0, 0)),
                   pl.BlockSpec((1, 1), lambda i: (0, 0))],
        out_shape=[jax.ShapeDtypeStruct((S, K), jnp.int32),
                   jax.ShapeDtypeStruct((NB, 1), jnp.int32),
                   jax.ShapeDtypeStruct((1, 1), jnp.int32)])(topi)


def _moe_scatter(h, pos2):
    mesh = plsc.VectorSubcoreMesh(core_axis_name="c", subcore_axis_name="s")

    def body(h_hbm, pos_hbm, xg_hbm, pos_v, rows_v, sem):
        wid = lax.axis_index("s") * 2 + lax.axis_index("c")
        t0 = lax.rem(wid, NSC // K) * JW
        pltpu.sync_copy(pos_hbm.at[wid], pos_v)
        pltpu.sync_copy(h_hbm.at[pl.ds(t0, JW)], rows_v)
        pltpu.async_copy(rows_v, xg_hbm.at[pos_v], sem).wait()

    call = pl.kernel(
        body, mesh=mesh,
        out_type=jax.ShapeDtypeStruct((NBP, D), _F32),
        scratch_types=[pltpu.VMEM((JW,), jnp.int32),
                       pltpu.VMEM((JW, D), _F32),
                       pltpu.SemaphoreType.DMA])
    return call(h, pos2)


def _moe_ffn(xg, w1, b1, w2, b2, be, nb):
    def body(be_s, nb_s, xg_ref, w1_ref, b1_ref, w2_ref, b2_ref, o_ref):
        b = pl.program_id(0)

        @pl.when(b < nb_s[0])
        def _():
            xb = xg_ref[...].astype(_BF)
            hh = jnp.dot(xb, w1_ref[0].astype(_BF),
                         preferred_element_type=_F32) + b1_ref[0]
            act = jax.nn.gelu(hh)
            o_ref[...] = jnp.dot(act.astype(_BF), w2_ref[0].astype(_BF),
                                 preferred_element_type=_F32) + b2_ref[0]

    grid_spec = pltpu.PrefetchScalarGridSpec(
        num_scalar_prefetch=2,
        grid=(NB,),
        in_specs=[
            pl.BlockSpec((BTM, D), lambda b, be, nb: (b, 0)),
            pl.BlockSpec((1, D, F), lambda b, be, nb: (be[b], 0, 0)),
            pl.BlockSpec((1, 1, F), lambda b, be, nb: (be[b], 0, 0)),
            pl.BlockSpec((1, F, D), lambda b, be, nb: (be[b], 0, 0)),
            pl.BlockSpec((1, 1, D), lambda b, be, nb: (be[b], 0, 0)),
        ],
        out_specs=pl.BlockSpec((BTM, D), lambda b, be, nb: (b, 0)),
    )
    return pl.pallas_call(
        body, grid_spec=grid_spec,
        out_shape=jax.ShapeDtypeStruct((NBP, D), _F32),
    )(be, nb, xg, w1, b1, w2, b2)


def _moe_gather(yg, pos2):
    mesh = plsc.VectorSubcoreMesh(core_axis_name="c", subcore_axis_name="s")

    def body(yg_hbm, pos_hbm, yy_hbm, pos_v, rows_v, sem):
        wid = lax.axis_index("s") * 2 + lax.axis_index("c")
        pltpu.sync_copy(pos_hbm.at[wid], pos_v)
        pltpu.async_copy(yg_hbm.at[pos_v], rows_v, sem).wait()
        pltpu.sync_copy(rows_v, yy_hbm.at[pl.ds(wid * JW, JW)])

    call = pl.kernel(
        body, mesh=mesh,
        out_type=jax.ShapeDtypeStruct((K * S, D), _F32),
        scratch_types=[pltpu.VMEM((JW,), jnp.int32),
                       pltpu.VMEM((JW, D), _F32),
                       pltpu.SemaphoreType.DMA])
    return call(yg, pos2)


def _combine_lnf(x1, y0, y1, tg, lng, lnb):
    def body(x1_ref, y0_ref, y1_ref, tg_ref, g_ref, b_ref, o_ref):
        tg = tg_ref[...]
        out = (x1_ref[...] + tg[:, 0:1] * y0_ref[...]
               + tg[:, 1:2] * y1_ref[...])
        o_ref[...] = _ln(out, g_ref[...], b_ref[...]).astype(_BF)

    xspec = pl.BlockSpec((BT, D), lambda t: (t, 0))
    vspec = pl.BlockSpec((1, D), lambda t: (0, 0))
    return pl.pallas_call(
        body, grid=(NT,),
        in_specs=[xspec, xspec, xspec,
                  pl.BlockSpec((BT, K), lambda t: (t, 0)), vspec, vspec],
        out_specs=xspec,
        out_shape=jax.ShapeDtypeStruct((S, D), _BF))(x1, y0, y1, tg,
                                                     lng, lnb)


def _logits(xf, wout):
    def body(x_ref, w_ref, o_ref):
        o_ref[...] = jnp.dot(x_ref[...], w_ref[...].astype(_BF),
                             preferred_element_type=_F32)

    return pl.pallas_call(
        body, grid=(NV,),
        in_specs=[pl.BlockSpec((S, D), lambda i: (0, 0)),
                  pl.BlockSpec((D, BV), lambda i: (0, i))],
        out_specs=pl.BlockSpec((S, BV), lambda i: (0, i)),
        out_shape=jax.ShapeDtypeStruct((S, V), _F32))(xf, wout)


def kernel(input_ids, params):
    p = params
    ids = input_ids.reshape(S).astype(jnp.int32)
    emb = _emb_gather(ids, p['tok_emb'])
    pre = ('add', p['pos_emb'])
    for l in range(L):
        q, k_, v, x = _qkv(emb if l == 0 else x1,
                           p['Wq'][l], p['bq'][l].reshape(1, D),
                           p['Wk'][l], p['bk'][l].reshape(1, D),
                           p['Wv'][l], p['bv'][l].reshape(1, D), pre=pre)
        attn = _attention(q, k_, v)
        x1, h, ti, tg = _oproj_lns(attn, p['Wo'][l],
                                   p['bo'][l].reshape(1, D), x,
                                   p['ln1_g'][l].reshape(1, D),
                                   p['ln1_b'][l].reshape(1, D),
                                   p['ln2_g'][l].reshape(1, D),
                                   p['ln2_b'][l].reshape(1, D),
                                   p['Wr'][l], p['br'][l].reshape(1, E))
        pos, be, nb = _route_meta(ti)
        pos2 = pos.T.reshape(NSC, JW)
        xg = _moe_scatter(h, pos2)
        yg = _moe_ffn(xg, p['W1'][l], p['b1'][l].reshape(E, 1, F),
                      p['W2'][l], p['b2'][l].reshape(E, 1, D),
                      be.reshape(NB), nb.reshape(1))
        yy = _moe_gather(yg, pos2)
        pre = ('combine', yy[:S], yy[S:], tg)
    xf = _combine_lnf(x1, yy[:S], yy[S:], tg,
                      p['lnf_g'].reshape(1, D), p['lnf_b'].reshape(1, D))
    logits = _logits(xf, p['Wout'])
    return logits.reshape(B, S, V)


# speedup vs baseline: 1.1157x; 1.1157x over previous
import jax
import jax.numpy as jnp
from jax import lax
from jax.experimental import pallas as pl
from jax.experimental.pallas import tpu as pltpu
from jax.experimental.pallas import tpu_sc as plsc

B, S, D, H, E, K, L, V, F = 1, 2048, 768, 12, 8, 2, 2, 32000, 1536
DH = D // H
BT = 512
NT = S // BT
BV = 1280
NV = V // BV
NSC = 32
TPW = S // NSC
BTM = 256
NB = 24
NBP = NB * BTM
JW = (K * S) // NSC

_BF = jnp.bfloat16
_F32 = jnp.float32


def _emb_gather(ids, table):
    mesh = plsc.VectorSubcoreMesh(core_axis_name="c", subcore_axis_name="s")

    def body(ids_hbm, table_hbm, out_hbm, idx_v, rows_v, sem):
        wid = lax.axis_index("s") * 2 + lax.axis_index("c")
        base = wid * TPW
        pltpu.sync_copy(ids_hbm.at[pl.ds(base, TPW)], idx_v)
        pltpu.async_copy(table_hbm.at[idx_v], rows_v, sem).wait()
        pltpu.sync_copy(rows_v, out_hbm.at[pl.ds(base, TPW)])

    call = pl.kernel(
        body,
        mesh=mesh,
        out_type=jax.ShapeDtypeStruct((S, D), _F32),
        scratch_types=[
            pltpu.VMEM((TPW,), jnp.int32),
            pltpu.VMEM((TPW, D), _F32),
            pltpu.SemaphoreType.DMA,
        ],
    )
    return call(ids, table)


def _ln(x, g, b):
    mu = jnp.mean(x, axis=-1, keepdims=True)
    var = jnp.mean((x - mu) ** 2, axis=-1, keepdims=True)
    return (x - mu) * lax.rsqrt(var + 1e-5) * g + b


def _qkv(x, wq, bq, wk, bk, wv, bv, pre=None):
    mode = pre[0] if pre is not None else None

    def body(*refs):
        if mode == 'add':
            (x_ref, p_ref, wq_ref, bq_ref, wk_ref, bk_ref, wv_ref, bv_ref,
             q_ref, k_ref, v_ref, xs_ref) = refs
            xs = x_ref[...] + p_ref[...]
        elif mode == 'combine':
            (x_ref, y0_ref, y1_ref, tg_ref, wq_ref, bq_ref, wk_ref,
             bk_ref, wv_ref, bv_ref, q_ref, k_ref, v_ref, xs_ref) = refs
            tg = tg_ref[...]
            xs = (x_ref[...] + tg[:, 0:1] * y0_ref[...]
                  + tg[:, 1:2] * y1_ref[...])
        else:
            (x_ref, wq_ref, bq_ref, wk_ref, bk_ref, wv_ref, bv_ref,
             q_ref, k_ref, v_ref, xs_ref) = refs
            xs = x_ref[...]
        xs_ref[...] = xs
        xb = xs.astype(_BF)
        for w_ref, b_ref, o_ref in ((wq_ref, bq_ref, q_ref),
                                    (wk_ref, bk_ref, k_ref),
                                    (wv_ref, bv_ref, v_ref)):
            o_ref[...] = jnp.dot(xb, w_ref[...].astype(_BF),
                                 preferred_element_type=_F32) + b_ref[...]

    xspec = pl.BlockSpec((BT, D), lambda t: (t, 0))
    wspec = pl.BlockSpec((D, D), lambda t: (0, 0))
    bspec = pl.BlockSpec((1, D), lambda t: (0, 0))
    in_specs = [xspec]
    args = [x]
    if mode == 'add':
        in_specs += [xspec]
        args += [pre[1]]
    elif mode == 'combine':
        in_specs += [xspec, xspec, pl.BlockSpec((BT, K), lambda t: (t, 0))]
        args += list(pre[1:])
    in_specs += [wspec, bspec, wspec, bspec, wspec, bspec]
    args += [wq, bq, wk, bk, wv, bv]
    return pl.pallas_call(
        body, grid=(NT,),
        in_specs=in_specs,
        out_specs=[xspec, xspec, xspec, xspec],
        out_shape=[jax.ShapeDtypeStruct((S, D), _F32)] * 4,
    )(*args)


def _attention(q, k, v):
    NH = 4

    def body(q_ref, k_ref, v_ref, o_ref):
        t = pl.program_id(1)
        ri = lax.broadcasted_iota(jnp.int32, (BT, BT), 0)
        ci = lax.broadcasted_iota(jnp.int32, (BT, BT), 1)
        diag_ok = ci <= ri

        m0 = jnp.full((BT, 1), -1e30, _F32)
        l0 = jnp.zeros((BT, 1), _F32)
        a0 = jnp.zeros((BT, DH), _F32)

        qb = (q_ref[...] * 0.125).astype(_BF)

        def step(c, carry):
            off_diag = c != t
            out = []
            for i in range(NH):
                m, l, acc = carry[i]
                kc = k_ref[pl.ds(c * BT, BT),
                           i * DH:(i + 1) * DH].astype(_BF)
                vc = v_ref[pl.ds(c * BT, BT),
                           i * DH:(i + 1) * DH].astype(_BF)
                s = lax.dot_general(qb[:, i * DH:(i + 1) * DH], kc,
                                    (((1,), (1,)), ((), ())),
                                    preferred_element_type=_F32)
                s = jnp.where(jnp.logical_or(off_diag, diag_ok), s, -1e9)
                mn = jnp.maximum(m, jnp.max(s, axis=-1, keepdims=True))
                p = jnp.exp(s - mn)
                corr = jnp.exp(m - mn)
                ln = l * corr + jnp.sum(p, axis=-1, keepdims=True)
                accn = acc * corr + jnp.dot(p.astype(_BF), vc,
                                            preferred_element_type=_F32)
                out.append((mn, ln, accn))
            return tuple(out)

        carry = lax.fori_loop(0, t + 1, step, tuple((m0, l0, a0)
                                                    for _ in range(NH)))
        o_ref[...] = jnp.concatenate([acc / l for (m, l, acc) in carry],
                                     axis=1)

    return pl.pallas_call(
        body, grid=(H // NH, NT),
        in_specs=[pl.BlockSpec((BT, NH * DH), lambda h, t: (t, h)),
                  pl.BlockSpec((S, NH * DH), lambda h, t: (0, h)),
                  pl.BlockSpec((S, NH * DH), lambda h, t: (0, h))],
        out_specs=pl.BlockSpec((BT, NH * DH), lambda h, t: (t, h)),
        out_shape=jax.ShapeDtypeStruct((S, D), _F32))(q, k, v)


def _oproj_lns(attn, wo, bo, x, g1, b1, g2, b2, wr, br):
    def body(a_ref, wo_ref, bo_ref, x_ref, g1_ref, b1_ref, g2_ref, b2_ref,
             wr_ref, br_ref, x1_ref, h_ref, ti_ref, tg_ref):
        a = jnp.dot(a_ref[...].astype(_BF), wo_ref[...].astype(_BF),
                    preferred_element_type=_F32)
        a = a + bo_ref[...] + x_ref[...]
        x1 = _ln(a, g1_ref[...], b1_ref[...])
        x1_ref[...] = x1
        hh = _ln(x1, g2_ref[...], b2_ref[...])
        h_ref[...] = hh
        rl = jnp.dot(hh, wr_ref[...],
                     preferred_element_type=_F32) + br_ref[...]
        iota = lax.broadcasted_iota(jnp.int32, (BT, E), 1)
        v1 = jnp.max(rl, axis=-1, keepdims=True)
        i1 = jnp.min(jnp.where(rl == v1, iota, E), axis=-1, keepdims=True)
        m1 = iota == i1
        rl2 = jnp.where(m1, -jnp.inf, rl)
        v2 = jnp.max(rl2, axis=-1, keepdims=True)
        i2 = jnp.min(jnp.where(rl2 == v2, iota, E), axis=-1, keepdims=True)
        m2 = iota == i2
        z = jnp.exp(v2 - v1)
        gate1 = 1.0 / (1.0 + z)
        gate2 = 1.0 - gate1
        ti_ref[...] = jnp.concatenate([i1, i2], axis=1)
        tg_ref[...] = jnp.concatenate([gate1, gate2], axis=1)

    xspec = pl.BlockSpec((BT, D), lambda t: (t, 0))
    wspec = pl.BlockSpec((D, D), lambda t: (0, 0))
    vspec = pl.BlockSpec((1, D), lambda t: (0, 0))
    tspec = pl.BlockSpec((BT, K), lambda t: (t, 0))
    return pl.pallas_call(
        body, grid=(NT,),
        in_specs=[xspec, wspec, vspec, xspec, vspec, vspec, vspec, vspec,
                  pl.BlockSpec((D, E), lambda t: (0, 0)),
                  pl.BlockSpec((1, E), lambda t: (0, 0))],
        out_specs=[xspec, xspec, tspec, tspec],
        out_shape=[jax.ShapeDtypeStruct((S, D), _F32),
                   jax.ShapeDtypeStruct((S, D), _F32),
                   jax.ShapeDtypeStruct((S, K), jnp.int32),
                   jax.ShapeDtypeStruct((S, K), _F32)],
    )(attn, wo, bo, x, g1, b1, g2, b2, wr, br)


def _route_meta(topi):
    CH = 512

    def body(ti_ref, pos_ref, be_ref, nb_ref):
        ti = ti_ref[...]
        lane_e = lax.broadcasted_iota(jnp.int32, (S, E), 1)
        oh = [jnp.where(lane_e == ti[:, kk:kk + 1], 1.0, 0.0)
              for kk in range(K)]
        lt = (lax.broadcasted_iota(jnp.int32, (CH, CH), 1) <
              lax.broadcasted_iota(jnp.int32, (CH, CH), 0)).astype(_F32)
        carry = jnp.zeros((1, E), _F32)
        ranks = []
        for kk in range(K):
            rk = []
            for ch in range(S // CH):
                blk = oh[kk][ch * CH:(ch + 1) * CH]
                rk.append(jnp.dot(lt, blk, preferred_element_type=_F32)
                          + carry)
                carry = carry + jnp.sum(blk, axis=0, keepdims=True)
            ranks.append(jnp.concatenate(rk, axis=0))
        counts = carry
        nblk = jnp.floor((counts + (BTM - 1)) / BTM)
        ut8 = (lax.broadcasted_iota(jnp.int32, (E, E), 0) <=
               lax.broadcasted_iota(jnp.int32, (E, E), 1)).astype(_F32)
        cumblk = jnp.dot(nblk, ut8, preferred_element_type=_F32)
        pad_off = (cumblk - nblk) * BTM
        poss = [jnp.sum(oh[kk] * (pad_off + ranks[kk]), axis=1,
                        keepdims=True) for kk in range(K)]
        pos_ref[...] = jnp.concatenate(poss, axis=1).astype(jnp.int32)
        cumblk_i = cumblk.astype(jnp.int32)
        rows_b = lax.broadcasted_iota(jnp.int32, (NB, E), 0)
        be = jnp.sum(jnp.where(rows_b >= cumblk_i, 1, 0), axis=1,
                     keepdims=True)
        be_ref[...] = jnp.minimum(be, E - 1)
        nb_ref[...] = jnp.sum(
            jnp.where(lax.broadcasted_iota(jnp.int32, (1, E), 1) == E - 1,
                      cumblk_i, 0), axis=1, keepdims=True)

    return pl.pallas_call(
        body, grid=(1,),
        in_specs=[pl.BlockSpec((S, K), lambda i: (0, 0))],
        out_specs=[pl.BlockSpec((S, K), lambda i: (0, 0)),
                   pl.BlockSpec((NB, 1), lambda i: (0, 0)),
                   pl.BlockSpec((1, 1), lambda i: (0, 0))],
        out_shape=[jax.ShapeDtypeStruct((S, K), jnp.int32),
                   jax.ShapeDtypeStruct((NB, 1), jnp.int32),
                   jax.ShapeDtypeStruct((1, 1), jnp.int32)])(topi)


def _moe_scatter(h, pos2):
    mesh = plsc.VectorSubcoreMesh(core_axis_name="c", subcore_axis_name="s")

    def body(h_hbm, pos_hbm, xg_hbm, pos_v, rows_v, sem):
        wid = lax.axis_index("s") * 2 + lax.axis_index("c")
        t0 = lax.rem(wid, NSC // K) * JW
        pltpu.sync_copy(pos_hbm.at[wid], pos_v)
        pltpu.sync_copy(h_hbm.at[pl.ds(t0, JW)], rows_v)
        pltpu.async_copy(rows_v, xg_hbm.at[pos_v], sem).wait()

    call = pl.kernel(
        body, mesh=mesh,
        out_type=jax.ShapeDtypeStruct((NBP, D), _F32),
        scratch_types=[pltpu.VMEM((JW,), jnp.int32),
                       pltpu.VMEM((JW, D), _F32),
                       pltpu.SemaphoreType.DMA])
    return call(h, pos2)


def _moe_ffn(xg, w1, b1, w2, b2, be, nb):
    def body(be_s, nb_s, xg_ref, w1_ref, b1_ref, w2_ref, b2_ref, o_ref):
        b = pl.program_id(0)

        @pl.when(b < nb_s[0])
        def _():
            xb = xg_ref[...].astype(_BF)
            hh = jnp.dot(xb, w1_ref[0].astype(_BF),
                         preferred_element_type=_F32) + b1_ref[0]
            act = jax.nn.gelu(hh)
            o_ref[...] = jnp.dot(act.astype(_BF), w2_ref[0].astype(_BF),
                                 preferred_element_type=_F32) + b2_ref[0]

    grid_spec = pltpu.PrefetchScalarGridSpec(
        num_scalar_prefetch=2,
        grid=(NB,),
        in_specs=[
            pl.BlockSpec((BTM, D), lambda b, be, nb: (b, 0)),
            pl.BlockSpec((1, D, F), lambda b, be, nb: (be[b], 0, 0)),
            pl.BlockSpec((1, 1, F), lambda b, be, nb: (be[b], 0, 0)),
            pl.BlockSpec((1, F, D), lambda b, be, nb: (be[b], 0, 0)),
            pl.BlockSpec((1, 1, D), lambda b, be, nb: (be[b], 0, 0)),
        ],
        out_specs=pl.BlockSpec((BTM, D), lambda b, be, nb: (b, 0)),
    )
    return pl.pallas_call(
        body, grid_spec=grid_spec,
        out_shape=jax.ShapeDtypeStruct((NBP, D), _F32),
    )(be, nb, xg, w1, b1, w2, b2)


def _moe_gather(yg, pos2):
    mesh = plsc.VectorSubcoreMesh(core_axis_name="c", subcore_axis_name="s")

    def body(yg_hbm, pos_hbm, yy_hbm, pos_v, rows_v, sem):
        wid = lax.axis_index("s") * 2 + lax.axis_index("c")
        pltpu.sync_copy(pos_hbm.at[wid], pos_v)
        pltpu.async_copy(yg_hbm.at[pos_v], rows_v, sem).wait()
        pltpu.sync_copy(rows_v, yy_hbm.at[pl.ds(wid * JW, JW)])

    call = pl.kernel(
        body, mesh=mesh,
        out_type=jax.ShapeDtypeStruct((K * S, D), _F32),
        scratch_types=[pltpu.VMEM((JW,), jnp.int32),
                       pltpu.VMEM((JW, D), _F32),
                       pltpu.SemaphoreType.DMA])
    return call(yg, pos2)


def _combine_lnf(x1, y0, y1, tg, lng, lnb):
    def body(x1_ref, y0_ref, y1_ref, tg_ref, g_ref, b_ref, o_ref):
        tg = tg_ref[...]
        out = (x1_ref[...] + tg[:, 0:1] * y0_ref[...]
               + tg[:, 1:2] * y1_ref[...])
        o_ref[...] = _ln(out, g_ref[...], b_ref[...]).astype(_BF)

    xspec = pl.BlockSpec((BT, D), lambda t: (t, 0))
    vspec = pl.BlockSpec((1, D), lambda t: (0, 0))
    return pl.pallas_call(
        body, grid=(NT,),
        in_specs=[xspec, xspec, xspec,
                  pl.BlockSpec((BT, K), lambda t: (t, 0)), vspec, vspec],
        out_specs=xspec,
        out_shape=jax.ShapeDtypeStruct((S, D), _BF))(x1, y0, y1, tg,
                                                     lng, lnb)


def _logits(xf, wout):
    def body(x_ref, w_ref, o_ref):
        o_ref[...] = jnp.dot(x_ref[...], w_ref[...].astype(_BF),
                             preferred_element_type=_F32)

    return pl.pallas_call(
        body, grid=(NV,),
        in_specs=[pl.BlockSpec((S, D), lambda i: (0, 0)),
                  pl.BlockSpec((D, BV), lambda i: (0, i))],
        out_specs=pl.BlockSpec((S, BV), lambda i: (0, i)),
        out_shape=jax.ShapeDtypeStruct((S, V), _F32))(xf, wout)


def kernel(input_ids, params):
    p = params
    ids = input_ids.reshape(S).astype(jnp.int32)
    emb = _emb_gather(ids, p['tok_emb'])
    pre = ('add', p['pos_emb'])
    for l in range(L):
        q, k_, v, x = _qkv(emb if l == 0 else x1,
                           p['Wq'][l], p['bq'][l].reshape(1, D),
                           p['Wk'][l], p['bk'][l].reshape(1, D),
                           p['Wv'][l], p['bv'][l].reshape(1, D), pre=pre)
        attn = _attention(q, k_, v)
        x1, h, ti, tg = _oproj_lns(attn, p['Wo'][l],
                                   p['bo'][l].reshape(1, D), x,
                                   p['ln1_g'][l].reshape(1, D),
                                   p['ln1_b'][l].reshape(1, D),
                                   p['ln2_g'][l].reshape(1, D),
                                   p['ln2_b'][l].reshape(1, D),
                                   p['Wr'][l], p['br'][l].reshape(1, E))
        pos, be, nb = _route_meta(ti)
        pos2 = pos.T.reshape(NSC, JW)
        xg = _moe_scatter(h, pos2)
        yg = _moe_ffn(xg, p['W1'][l], p['b1'][l].reshape(E, 1, F),
                      p['W2'][l], p['b2'][l].reshape(E, 1, D),
                      be.reshape(NB), nb.reshape(1))
        yy = _moe_gather(yg, pos2)
        pre = ('combine', yy[:S], yy[S:], tg)
    xf = _combine_lnf(x1, yy[:S], yy[S:], tg,
                      p['lnf_g'].reshape(1, D), p['lnf_b'].reshape(1, D))
    logits = _logits(xf, p['Wout'])
    return logits.reshape(B, S, V)
